# Initial kernel scaffold; baseline (speedup 1.0000x reference)
import jax
import jax.numpy as jnp
from jax.experimental import pallas as pl


def kernel(x, edge_index, W1, b1, W2, b2):
    raise NotImplementedError("write your pallas kernel here")


# speedup vs baseline: 5.4578x; 5.4578x over previous
import jax
import jax.numpy as jnp
from jax import lax
from jax.experimental import pallas as pl
from jax.experimental.pallas import tpu as pltpu
from jax.experimental.pallas import tpu_sc as plsc

N = 10000
C = 128
NC = 2
NS = 16
L = 16
NW = NC * NS
K = 128
CH = 80
G = K // L
EPT = K * CH
EPAD = NW * EPT
NPAD = 10240
RPT = NPAD // NS
DUMMY = N
BT = 2000


def _sc_mesh():
    return plsc.VectorSubcoreMesh(
        core_axis_name="c", subcore_axis_name="s",
        num_cores=NC, num_subcores=NS)


def _deg_body(dst_hbm, zeros_hbm, ones_hbm, out_hbm, dst_v, ones_v, deg_sh):
    cid = lax.axis_index("c")
    sid = lax.axis_index("s")
    wid = sid * NC + cid

    pltpu.sync_copy(ones_hbm, ones_v)
    pltpu.sync_copy(zeros_hbm.at[pl.ds(sid * RPT, RPT)],
                    deg_sh.at[pl.ds(sid * RPT, RPT)])
    plsc.subcore_barrier()

    pltpu.sync_copy(dst_hbm.at[wid], dst_v)

    def chunk(t, _):
        r = t // G
        c = t % G
        idx_vec = dst_v[r, pl.ds(c * L, L)]
        pltpu.sync_copy(ones_v, deg_sh.at[idx_vec], add=True)
        return 0
    lax.fori_loop(0, CH * G, chunk, 0)

    plsc.subcore_barrier()
    pltpu.sync_copy(deg_sh.at[pl.ds(sid * RPT, RPT)],
                    out_hbm.at[cid, pl.ds(sid * RPT, RPT)])


def _deg_call(dst_r):
    zeros = jnp.zeros((NPAD, C), jnp.float32)
    ones = jnp.ones((L, C), jnp.float32)
    return pl.kernel(
        _deg_body,
        out_type=jax.ShapeDtypeStruct((NC, NPAD, C), jnp.float32),
        mesh=_sc_mesh(),
        scratch_types=[
            pltpu.VMEM((CH, K), jnp.int32),
            pltpu.VMEM((L, C), jnp.float32),
            pltpu.VMEM_SHARED((NPAD, C), jnp.float32),
        ],
    )(dst_r, zeros, ones)


def _agg_body(g_hbm, src_hbm, dst_hbm, zeros_hbm, out_hbm,
              src_v, dst_v, rows_v, agg_sh, sem):
    cid = lax.axis_index("c")
    sid = lax.axis_index("s")
    wid = sid * NC + cid

    pltpu.sync_copy(zeros_hbm.at[pl.ds(sid * RPT, RPT)],
                    agg_sh.at[pl.ds(sid * RPT, RPT)])
    plsc.subcore_barrier()

    pltpu.sync_copy(src_hbm.at[wid], src_v)
    pltpu.sync_copy(dst_hbm.at[wid], dst_v)

    def chunk(t, _):
        r = t // G
        c = t % G
        sidx = src_v[r, pl.ds(c * L, L)]
        didx = dst_v[r, pl.ds(c * L, L)]
        pltpu.async_copy(g_hbm.at[sidx], rows_v, sem).wait()
        pltpu.sync_copy(rows_v, agg_sh.at[didx], add=True)
        return 0
    lax.fori_loop(0, CH * G, chunk, 0)

    plsc.subcore_barrier()
    pltpu.sync_copy(agg_sh.at[pl.ds(sid * RPT, RPT)],
                    out_hbm.at[cid, pl.ds(sid * RPT, RPT)])


def _agg_call(g, src_r, dst_r):
    zeros = jnp.zeros((NPAD, C), jnp.float32)
    return pl.kernel(
        _agg_body,
        out_type=jax.ShapeDtypeStruct((NC, NPAD, C), jnp.float32),
        mesh=_sc_mesh(),
        scratch_types=[
            pltpu.VMEM((CH, K), jnp.int32),
            pltpu.VMEM((CH, K), jnp.int32),
            pltpu.VMEM((L, C), jnp.float32),
            pltpu.VMEM_SHARED((NPAD, C), jnp.float32),
            pltpu.SemaphoreType.DMA,
        ],
    )(g, src_r, dst_r, zeros)


def _mm_body(x_ref, w_ref, o_ref):
    o_ref[...] = jnp.dot(x_ref[...], w_ref[...],
                         preferred_element_type=jnp.float32)


def _matmul(x, w):
    return pl.pallas_call(
        _mm_body,
        grid=(N // BT,),
        in_specs=[pl.BlockSpec((BT, C), lambda i: (i, 0)),
                  pl.BlockSpec((C, C), lambda i: (0, 0))],
        out_specs=pl.BlockSpec((BT, C), lambda i: (i, 0)),
        out_shape=jax.ShapeDtypeStruct((N, C), jnp.float32),
    )(x, w)


def _scale_body(p_ref, d0_ref, d1_ref, g_ref, dinv_ref):
    deg = d0_ref[...] + d1_ref[...] + 1.0
    dinvf = lax.rsqrt(deg)
    dinv_ref[...] = dinvf
    g_ref[...] = p_ref[...] * dinvf


def _scale(p1, d0, d1):
    return pl.pallas_call(
        _scale_body,
        grid=(N // BT,),
        in_specs=[pl.BlockSpec((BT, C), lambda i: (i, 0)),
                  pl.BlockSpec((BT, C), lambda i: (i, 0)),
                  pl.BlockSpec((BT, C), lambda i: (i, 0))],
        out_specs=[pl.BlockSpec((BT, C), lambda i: (i, 0)),
                   pl.BlockSpec((BT, C), lambda i: (i, 0))],
        out_shape=[jax.ShapeDtypeStruct((N, C), jnp.float32),
                   jax.ShapeDtypeStruct((N, C), jnp.float32)],
    )(p1, d0, d1)


def _layer2_body(a0_ref, a1_ref, g_ref, dinv_ref, b_ref, w_ref, o_ref):
    h = (a0_ref[...] + a1_ref[...] + g_ref[...]) * dinv_ref[...] + b_ref[...]
    h = jnp.maximum(h, 0.0)
    o_ref[...] = jnp.dot(h, w_ref[...],
                         preferred_element_type=jnp.float32) * dinv_ref[...]


def _layer2(a0, a1, g1, dinvf, b1, w2):
    return pl.pallas_call(
        _layer2_body,
        grid=(N // BT,),
        in_specs=[pl.BlockSpec((BT, C), lambda i: (i, 0)),
                  pl.BlockSpec((BT, C), lambda i: (i, 0)),
                  pl.BlockSpec((BT, C), lambda i: (i, 0)),
                  pl.BlockSpec((BT, C), lambda i: (i, 0)),
                  pl.BlockSpec((1, C), lambda i: (0, 0)),
                  pl.BlockSpec((C, C), lambda i: (0, 0))],
        out_specs=pl.BlockSpec((BT, C), lambda i: (i, 0)),
        out_shape=jax.ShapeDtypeStruct((N, C), jnp.float32),
    )(a0, a1, g1, dinvf, b1, w2)


def _final_body(a0_ref, a1_ref, g_ref, dinv_ref, b_ref, o_ref):
    o_ref[...] = ((a0_ref[...] + a1_ref[...] + g_ref[...]) * dinv_ref[...]
                  + b_ref[...])


def _final(a0, a1, g2, dinvf, b2):
    return pl.pallas_call(
        _final_body,
        grid=(N // BT,),
        in_specs=[pl.BlockSpec((BT, C), lambda i: (i, 0)),
                  pl.BlockSpec((BT, C), lambda i: (i, 0)),
                  pl.BlockSpec((BT, C), lambda i: (i, 0)),
                  pl.BlockSpec((BT, C), lambda i: (i, 0)),
                  pl.BlockSpec((1, C), lambda i: (0, 0))],
        out_specs=pl.BlockSpec((BT, C), lambda i: (i, 0)),
        out_shape=jax.ShapeDtypeStruct((N, C), jnp.float32),
    )(a0, a1, g2, dinvf, b2)


def kernel(x, edge_index, W1, b1, W2, b2):
    ei = edge_index.astype(jnp.int32)
    src, dst = ei[0], ei[1]
    pad = EPAD - src.shape[0]
    src_r = jnp.concatenate(
        [src, jnp.zeros((pad,), jnp.int32)]).reshape(NW, CH, K)
    dst_r = jnp.concatenate(
        [dst, jnp.full((pad,), DUMMY, jnp.int32)]).reshape(NW, CH, K)

    degp = _deg_call(dst_r)
    d0 = jnp.broadcast_to(degp[0, :N, 0:1], (N, C))
    d1 = jnp.broadcast_to(degp[1, :N, 0:1], (N, C))
    p1 = _matmul(x, W1)
    g1, dinvf = _scale(p1, d0, d1)

    a1 = _agg_call(g1, src_r, dst_r)
    g2 = _layer2(a1[0, :N], a1[1, :N], g1, dinvf,
                 b1.reshape(1, C), W2)

    a2 = _agg_call(g2, src_r, dst_r)
    return _final(a2[0, :N], a2[1, :N], g2, dinvf, b2.reshape(1, C))


# speedup vs baseline: 7.7464x; 1.4193x over previous
import jax
import jax.numpy as jnp
from jax import lax
from jax.experimental import pallas as pl
from jax.experimental.pallas import tpu as pltpu
from jax.experimental.pallas import tpu_sc as plsc

N = 10000
C = 128
NC = 2
NS = 16
L = 16
NW = NC * NS
K = 128
CH = 80
G = K // L
EPT = K * CH
EPAD = NW * EPT
NPAD = 10240
RPT = NPAD // NS
DUMMY = N
BT = 2000


def _sc_mesh():
    return plsc.VectorSubcoreMesh(
        core_axis_name="c", subcore_axis_name="s",
        num_cores=NC, num_subcores=NS)


def _deg_body(dst_hbm, zeros_hbm, ones_hbm, out_hbm, dst_v, ones_v, deg_sh):
    cid = lax.axis_index("c")
    sid = lax.axis_index("s")
    wid = sid * NC + cid

    pltpu.sync_copy(ones_hbm, ones_v)
    pltpu.sync_copy(zeros_hbm.at[pl.ds(sid * RPT, RPT)],
                    deg_sh.at[pl.ds(sid * RPT, RPT)])
    plsc.subcore_barrier()

    pltpu.sync_copy(dst_hbm.at[wid], dst_v)

    def chunk(t, _):
        r = t // G
        c = t % G
        idx_vec = dst_v[r, pl.ds(c * L, L)]
        pltpu.sync_copy(ones_v, deg_sh.at[idx_vec], add=True)
        return 0
    lax.fori_loop(0, CH * G, chunk, 0)

    plsc.subcore_barrier()
    pltpu.sync_copy(deg_sh.at[pl.ds(sid * RPT, RPT)],
                    out_hbm.at[cid, pl.ds(sid * RPT, RPT)])


def _deg_call(dst_r):
    zeros = jnp.zeros((NPAD, C), jnp.float32)
    ones = jnp.ones((L, C), jnp.float32)
    return pl.kernel(
        _deg_body,
        out_type=jax.ShapeDtypeStruct((NC, NPAD, C), jnp.float32),
        mesh=_sc_mesh(),
        scratch_types=[
            pltpu.VMEM((CH, K), jnp.int32),
            pltpu.VMEM((L, C), jnp.float32),
            pltpu.VMEM_SHARED((NPAD, C), jnp.float32),
        ],
    )(dst_r, zeros, ones)


def _agg_body(g_hbm, src_hbm, dst_hbm, zeros_hbm, out_hbm,
              src_v, dst_v, rows_v, agg_sh, sem):
    cid = lax.axis_index("c")
    sid = lax.axis_index("s")
    wid = sid * NC + cid

    pltpu.sync_copy(zeros_hbm.at[pl.ds(sid * RPT, RPT)],
                    agg_sh.at[pl.ds(sid * RPT, RPT)])
    plsc.subcore_barrier()

    pltpu.sync_copy(src_hbm.at[wid], src_v)
    pltpu.sync_copy(dst_hbm.at[wid], dst_v)

    def chunk(r, _):
        pltpu.async_copy(g_hbm.at[src_v.at[r]], rows_v, sem).wait()
        pltpu.sync_copy(rows_v, agg_sh.at[dst_v.at[r]], add=True)
        return 0
    lax.fori_loop(0, CH, chunk, 0)

    plsc.subcore_barrier()
    pltpu.sync_copy(agg_sh.at[pl.ds(sid * RPT, RPT)],
                    out_hbm.at[cid, pl.ds(sid * RPT, RPT)])


def _agg_call(g, src_r, dst_r):
    zeros = jnp.zeros((NPAD, C), jnp.float32)
    return pl.kernel(
        _agg_body,
        out_type=jax.ShapeDtypeStruct((NC, NPAD, C), jnp.float32),
        mesh=_sc_mesh(),
        scratch_types=[
            pltpu.VMEM((CH, K), jnp.int32),
            pltpu.VMEM((CH, K), jnp.int32),
            pltpu.VMEM((K, C), jnp.float32),
            pltpu.VMEM_SHARED((NPAD, C), jnp.float32),
            pltpu.SemaphoreType.DMA,
        ],
    )(g, src_r, dst_r, zeros)


def _mm_body(x_ref, w_ref, o_ref):
    o_ref[...] = jnp.dot(x_ref[...], w_ref[...],
                         preferred_element_type=jnp.float32)


def _matmul(x, w):
    return pl.pallas_call(
        _mm_body,
        grid=(N // BT,),
        in_specs=[pl.BlockSpec((BT, C), lambda i: (i, 0)),
                  pl.BlockSpec((C, C), lambda i: (0, 0))],
        out_specs=pl.BlockSpec((BT, C), lambda i: (i, 0)),
        out_shape=jax.ShapeDtypeStruct((N, C), jnp.float32),
    )(x, w)


def _scale_body(p_ref, d0_ref, d1_ref, g_ref, dinv_ref):
    deg = d0_ref[...] + d1_ref[...] + 1.0
    dinvf = lax.rsqrt(deg)
    dinv_ref[...] = dinvf
    g_ref[...] = p_ref[...] * dinvf


def _scale(p1, d0, d1):
    return pl.pallas_call(
        _scale_body,
        grid=(N // BT,),
        in_specs=[pl.BlockSpec((BT, C), lambda i: (i, 0)),
                  pl.BlockSpec((BT, C), lambda i: (i, 0)),
                  pl.BlockSpec((BT, C), lambda i: (i, 0))],
        out_specs=[pl.BlockSpec((BT, C), lambda i: (i, 0)),
                   pl.BlockSpec((BT, C), lambda i: (i, 0))],
        out_shape=[jax.ShapeDtypeStruct((N, C), jnp.float32),
                   jax.ShapeDtypeStruct((N, C), jnp.float32)],
    )(p1, d0, d1)


def _layer2_body(a0_ref, a1_ref, g_ref, dinv_ref, b_ref, w_ref, o_ref):
    h = (a0_ref[...] + a1_ref[...] + g_ref[...]) * dinv_ref[...] + b_ref[...]
    h = jnp.maximum(h, 0.0)
    o_ref[...] = jnp.dot(h, w_ref[...],
                         preferred_element_type=jnp.float32) * dinv_ref[...]


def _layer2(a0, a1, g1, dinvf, b1, w2):
    return pl.pallas_call(
        _layer2_body,
        grid=(N // BT,),
        in_specs=[pl.BlockSpec((BT, C), lambda i: (i, 0)),
                  pl.BlockSpec((BT, C), lambda i: (i, 0)),
                  pl.BlockSpec((BT, C), lambda i: (i, 0)),
                  pl.BlockSpec((BT, C), lambda i: (i, 0)),
                  pl.BlockSpec((1, C), lambda i: (0, 0)),
                  pl.BlockSpec((C, C), lambda i: (0, 0))],
        out_specs=pl.BlockSpec((BT, C), lambda i: (i, 0)),
        out_shape=jax.ShapeDtypeStruct((N, C), jnp.float32),
    )(a0, a1, g1, dinvf, b1, w2)


def _final_body(a0_ref, a1_ref, g_ref, dinv_ref, b_ref, o_ref):
    o_ref[...] = ((a0_ref[...] + a1_ref[...] + g_ref[...]) * dinv_ref[...]
                  + b_ref[...])


def _final(a0, a1, g2, dinvf, b2):
    return pl.pallas_call(
        _final_body,
        grid=(N // BT,),
        in_specs=[pl.BlockSpec((BT, C), lambda i: (i, 0)),
                  pl.BlockSpec((BT, C), lambda i: (i, 0)),
                  pl.BlockSpec((BT, C), lambda i: (i, 0)),
                  pl.BlockSpec((BT, C), lambda i: (i, 0)),
                  pl.BlockSpec((1, C), lambda i: (0, 0))],
        out_specs=pl.BlockSpec((BT, C), lambda i: (i, 0)),
        out_shape=jax.ShapeDtypeStruct((N, C), jnp.float32),
    )(a0, a1, g2, dinvf, b2)


def kernel(x, edge_index, W1, b1, W2, b2):
    ei = edge_index.astype(jnp.int32)
    src, dst = ei[0], ei[1]
    pad = EPAD - src.shape[0]
    src_r = jnp.concatenate(
        [src, jnp.zeros((pad,), jnp.int32)]).reshape(NW, CH, K)
    dst_r = jnp.concatenate(
        [dst, jnp.full((pad,), DUMMY, jnp.int32)]).reshape(NW, CH, K)

    degp = _deg_call(dst_r)
    d0 = jnp.broadcast_to(degp[0, :N, 0:1], (N, C))
    d1 = jnp.broadcast_to(degp[1, :N, 0:1], (N, C))
    p1 = _matmul(x, W1)
    g1, dinvf = _scale(p1, d0, d1)

    a1 = _agg_call(g1, src_r, dst_r)
    g2 = _layer2(a1[0, :N], a1[1, :N], g1, dinvf,
                 b1.reshape(1, C), W2)

    a2 = _agg_call(g2, src_r, dst_r)
    return _final(a2[0, :N], a2[1, :N], g2, dinvf, b2.reshape(1, C))
